# baseline (speedup 1.0000x reference)
import functools

import jax
import jax.numpy as jnp
from jax import lax
from jax.experimental import pallas as pl
from jax.experimental.pallas import tpu as pltpu
from jax.experimental.pallas import tpu_sc as plsc

N = 65536
OUT = 2 * N
NC = 2
NS = 16
NW = NC * NS
L = 16
C = OUT // NW
SCAN = N // NS
YB = C + L
SENT = N


def _sc_body(x_hbm, y_hbm, out_hbm, xscan, ybuf, obuf, minbuf, allbuf, pshared):
    c_id = lax.axis_index("c")
    s_id = lax.axis_index("s")
    wid = c_id * NS + s_id
    lane = lax.iota(jnp.int32, L)

    scan_base = s_id * SCAN
    pltpu.sync_copy(x_hbm.at[pl.ds(pl.multiple_of(scan_base, SCAN), SCAN)],
                    xscan)

    def scan_step(j, mv):
        v = xscan[pl.ds(j * L, L)]
        idx = scan_base + j * L + lane
        cand = jnp.where(jnp.abs(v) == jnp.inf, idx, jnp.int32(SENT))
        return jnp.minimum(mv, cand)

    mv = lax.fori_loop(0, SCAN // L, scan_step,
                       jnp.full((L,), SENT, jnp.int32))
    minbuf[...] = mv
    pltpu.sync_copy(minbuf, pshared.at[s_id])
    plsc.subcore_barrier()
    pltpu.sync_copy(pshared, allbuf)
    m = allbuf[0, :]
    for k in range(1, NS):
        m = jnp.minimum(m, allbuf[k, :])
    pmin = m[0]
    for k in range(1, L):
        pmin = jnp.minimum(pmin, m[k])
    p = jnp.where(pmin == jnp.int32(SENT), jnp.int32(0), pmin)

    s_out = wid * C
    q = s_out - p
    q_c = jnp.clip(q & ~(L - 1), 0, N - YB)
    r = q - q_c
    q_c = pl.multiple_of(q_c, L)
    pltpu.sync_copy(y_hbm.at[pl.ds(q_c, YB)], ybuf)

    p_hi = p + N
    infv = jnp.full((L,), jnp.inf, jnp.float32)

    def compose_step(j, carry):
        bl = j * L + lane
        i_vec = s_out + bl
        xv = xscan[pl.ds(j * L, L)]
        y_idx = jnp.clip(bl + r, 0, YB - 1)
        yv = plsc.load_gather(ybuf, [y_idx])
        res = jnp.where(i_vec < p, xv, jnp.where(i_vec < p_hi, yv, infv))
        obuf[pl.ds(j * L, L)] = res
        return carry

    lax.fori_loop(0, C // L, compose_step, jnp.int32(0))
    pltpu.sync_copy(obuf, out_hbm.at[pl.ds(pl.multiple_of(s_out, C), C)])


@jax.jit
def kernel(x, y):
    mesh = plsc.VectorSubcoreMesh(core_axis_name="c", subcore_axis_name="s")
    run = pl.kernel(
        _sc_body,
        out_type=jax.ShapeDtypeStruct((OUT,), jnp.float32),
        mesh=mesh,
        compiler_params=pltpu.CompilerParams(needs_layout_passes=False),
        scratch_types=[
            pltpu.VMEM((SCAN,), jnp.float32),
            pltpu.VMEM((YB,), jnp.float32),
            pltpu.VMEM((C,), jnp.float32),
            pltpu.VMEM((L,), jnp.int32),
            pltpu.VMEM((NS, L), jnp.int32),
            pltpu.VMEM_SHARED((NS, L), jnp.int32),
        ],
    )
    return run(x, y)


# speedup vs baseline: 1.0420x; 1.0420x over previous
import functools

import jax
import jax.numpy as jnp
from jax import lax
from jax.experimental import pallas as pl
from jax.experimental.pallas import tpu as pltpu
from jax.experimental.pallas import tpu_sc as plsc

N = 65536
OUT = 2 * N
NC = 2
NS = 16
NW = NC * NS
L = 16
C = OUT // NW
SCAN = N // NS
YB = C + L
SENT = N


def _sc_body(x_hbm, y_hbm, out_hbm, xscan, ybuf, obuf, minbuf, allbuf, pshared):
    c_id = lax.axis_index("c")
    s_id = lax.axis_index("s")
    wid = c_id * NS + s_id
    lane = lax.iota(jnp.int32, L)

    scan_base = s_id * SCAN
    pltpu.sync_copy(x_hbm.at[pl.ds(pl.multiple_of(scan_base, SCAN), SCAN)],
                    xscan)

    @plsc.parallel_loop(0, SCAN, step=L, unroll=8,
                        carry=jnp.full((L,), SENT, jnp.int32))
    def mv(i, acc):
        v = xscan[pl.ds(i, L)]
        idx = scan_base + i + lane
        cand = jnp.where(jnp.abs(v) == jnp.inf, idx, jnp.int32(SENT))
        return jnp.minimum(acc, cand)
    minbuf[...] = mv
    pltpu.sync_copy(minbuf, pshared.at[s_id])
    plsc.subcore_barrier()
    pltpu.sync_copy(pshared, allbuf)
    m = allbuf[0, :]
    for k in range(1, NS):
        m = jnp.minimum(m, allbuf[k, :])
    pmin = m[0]
    for k in range(1, L):
        pmin = jnp.minimum(pmin, m[k])
    p = jnp.where(pmin == jnp.int32(SENT), jnp.int32(0), pmin)

    s_out = wid * C
    q = s_out - p
    q_c = jnp.clip(q & ~(L - 1), 0, N - YB)
    r = q - q_c
    q_c = pl.multiple_of(q_c, L)
    pltpu.sync_copy(y_hbm.at[pl.ds(q_c, YB)], ybuf)

    p_hi = p + N
    infv = jnp.full((L,), jnp.inf, jnp.float32)

    @plsc.parallel_loop(0, C, step=L, unroll=8)
    def _(i):
        bl = i + lane
        i_vec = s_out + bl
        xv = xscan[pl.ds(i, L)]
        y_idx = jnp.clip(bl + r, 0, YB - 1)
        yv = plsc.load_gather(ybuf, [y_idx])
        obuf[pl.ds(i, L)] = jnp.where(
            i_vec < p, xv, jnp.where(i_vec < p_hi, yv, infv))
    pltpu.sync_copy(obuf, out_hbm.at[pl.ds(pl.multiple_of(s_out, C), C)])


@jax.jit
def kernel(x, y):
    mesh = plsc.VectorSubcoreMesh(core_axis_name="c", subcore_axis_name="s")
    run = pl.kernel(
        _sc_body,
        out_type=jax.ShapeDtypeStruct((OUT,), jnp.float32),
        mesh=mesh,
        compiler_params=pltpu.CompilerParams(needs_layout_passes=False),
        scratch_types=[
            pltpu.VMEM((SCAN,), jnp.float32),
            pltpu.VMEM((YB,), jnp.float32),
            pltpu.VMEM((C,), jnp.float32),
            pltpu.VMEM((L,), jnp.int32),
            pltpu.VMEM((NS, L), jnp.int32),
            pltpu.VMEM_SHARED((NS, L), jnp.int32),
        ],
    )
    return run(x, y)


# speedup vs baseline: 1.0877x; 1.0438x over previous
import jax
import jax.numpy as jnp
from jax import lax
from jax.experimental import pallas as pl
from jax.experimental.pallas import tpu as pltpu
from jax.experimental.pallas import tpu_sc as plsc

N = 65536
OUT = 2 * N
NC = 2
NS = 16
NW = NC * NS
L = 16
C = OUT // NW
SCAN = N // NS
YB = C + L
SENT = N


def _sc_body(x_hbm, y_hbm, out_hbm, xscan, ybuf, obuf, minbuf, allbuf, pshared):
    c_id = lax.axis_index("c")
    s_id = lax.axis_index("s")
    wid = c_id * NS + s_id
    lane = lax.iota(jnp.int32, L)

    scan_base = s_id * SCAN
    pltpu.sync_copy(x_hbm.at[pl.ds(pl.multiple_of(scan_base, SCAN), SCAN)],
                    xscan)

    @plsc.parallel_loop(0, SCAN, step=L, unroll=8,
                        carry=jnp.full((L,), SENT, jnp.int32))
    def mv(i, acc):
        v = xscan[pl.ds(i, L)]
        cand = jnp.where(jnp.abs(v) == jnp.inf, i, jnp.int32(SENT))
        return jnp.minimum(acc, cand)

    minbuf[...] = jnp.minimum(scan_base + mv + lane, jnp.int32(SENT))
    pltpu.sync_copy(minbuf, pshared.at[s_id])
    plsc.subcore_barrier()
    pltpu.sync_copy(pshared, allbuf)
    m = allbuf[0, :]
    for k in range(1, NS):
        m = jnp.minimum(m, allbuf[k, :])
    pmin = m[0]
    for k in range(1, L):
        pmin = jnp.minimum(pmin, m[k])
    p = jnp.where(pmin == jnp.int32(SENT), jnp.int32(0), pmin)
    p_hi = p + N

    s_out = wid * C
    q = s_out - p
    is_x = s_out + C <= p
    is_y = ((q >= 0) & (q + C <= N) & ((q & (L - 1)) == 0)
            & jnp.logical_not(is_x))
    is_inf = q >= N
    is_gen = jnp.logical_not(is_x | is_y | is_inf)
    out_chunk = out_hbm.at[pl.ds(pl.multiple_of(s_out, C), C)]

    @pl.when(is_x)
    def _():
        pltpu.sync_copy(xscan, out_chunk)

    @pl.when(is_y)
    def _():
        qa = pl.multiple_of(jnp.clip(q, 0, N - C), L)
        pltpu.sync_copy(y_hbm.at[pl.ds(qa, C)], ybuf.at[pl.ds(0, C)])
        pltpu.sync_copy(ybuf.at[pl.ds(0, C)], out_chunk)

    @pl.when(is_inf)
    def _():
        infv = jnp.full((L,), jnp.inf, jnp.float32)

        @plsc.parallel_loop(0, C, step=L, unroll=8)
        def _(i):
            obuf[pl.ds(i, L)] = infv

        pltpu.sync_copy(obuf, out_chunk)

    @pl.when(is_gen)
    def _():
        q_c = pl.multiple_of(jnp.clip(q & ~(L - 1), 0, N - YB), L)
        r = q - q_c
        pltpu.sync_copy(y_hbm.at[pl.ds(q_c, YB)], ybuf)
        infv = jnp.full((L,), jnp.inf, jnp.float32)

        @plsc.parallel_loop(0, C, step=L, unroll=8)
        def _(i):
            bl = i + lane
            i_vec = s_out + bl
            xv = xscan[pl.ds(i, L)]
            y_idx = jnp.clip(bl + r, 0, YB - 1)
            yv = plsc.load_gather(ybuf, [y_idx])
            obuf[pl.ds(i, L)] = jnp.where(
                i_vec < p, xv, jnp.where(i_vec < p_hi, yv, infv))

        pltpu.sync_copy(obuf, out_chunk)


@jax.jit
def kernel(x, y):
    mesh = plsc.VectorSubcoreMesh(core_axis_name="c", subcore_axis_name="s")
    run = pl.kernel(
        _sc_body,
        out_type=jax.ShapeDtypeStruct((OUT,), jnp.float32),
        mesh=mesh,
        compiler_params=pltpu.CompilerParams(needs_layout_passes=False),
        scratch_types=[
            pltpu.VMEM((SCAN,), jnp.float32),
            pltpu.VMEM((YB,), jnp.float32),
            pltpu.VMEM((C,), jnp.float32),
            pltpu.VMEM((L,), jnp.int32),
            pltpu.VMEM((NS, L), jnp.int32),
            pltpu.VMEM_SHARED((NS, L), jnp.int32),
        ],
    )
    return run(x, y)
